# Initial kernel scaffold; baseline (speedup 1.0000x reference)
import jax
import jax.numpy as jnp
from jax.experimental import pallas as pl


def kernel(seq, seg, token_table, position_table, segment_table):
    raise NotImplementedError("write your pallas kernel here")


# speedup vs baseline: 1.2716x; 1.2716x over previous
import functools

import jax
import jax.numpy as jnp
from jax import lax
from jax.experimental import pallas as pl
from jax.experimental.pallas import tpu as pltpu
from jax.experimental.pallas import tpu_sc as plsc

_BATCH = 64
_SENT = 512
_HID = 768
_VOCAB = 100000
_SEGS = 2

_N = _BATCH * _SENT
_NW = 32
_PER_W = _N // _NW
_W = 64
_CHUNKS = _PER_W // _W
_LANES = 16
_HSL = _HID // _LANES


def _emb_kernel(seq_hbm, psg_idx_hbm, tok_tab, psg_tab, out_hbm,
                idx_v, pidx_v, tok_v, acc_v, sem_t, sem_p):
    wid = lax.axis_index("s") * 2 + lax.axis_index("c")
    base = wid * _PER_W

    for c in range(_CHUNKS):
        off = base + c * _W

        pltpu.sync_copy(seq_hbm.at[pl.ds(off, _W)], idx_v)
        pltpu.sync_copy(psg_idx_hbm.at[pl.ds(off, _W)], pidx_v)

        cp_t = pltpu.async_copy(tok_tab.at[idx_v], tok_v, sem_t)
        cp_p = pltpu.async_copy(psg_tab.at[pidx_v], acc_v, sem_p)
        cp_t.wait()
        cp_p.wait()

        def _add_row(j, _):
            for h in range(_HSL):
                sl = pl.ds(h * _LANES, _LANES)
                acc_v[j, sl] = acc_v[j, sl] + tok_v[j, sl]
            return 0

        lax.fori_loop(0, _W, _add_row, 0)

        pltpu.sync_copy(acc_v, out_hbm.at[pl.ds(off, _W)])


@jax.jit
def _emb(seq_flat, psg_idx, token_table, psg_table):
    mesh = plsc.VectorSubcoreMesh(core_axis_name="c", subcore_axis_name="s")
    kfn = pl.kernel(
        _emb_kernel,
        out_type=jax.ShapeDtypeStruct((_N, _HID), jnp.float32),
        mesh=mesh,
        scratch_types=[
            pltpu.VMEM((_W,), jnp.int32),
            pltpu.VMEM((_W,), jnp.int32),
            pltpu.VMEM((_W, _HID), jnp.float32),
            pltpu.VMEM((_W, _HID), jnp.float32),
            pltpu.SemaphoreType.DMA,
            pltpu.SemaphoreType.DMA,
        ],
    )
    return kfn(seq_flat, psg_idx, token_table, psg_table)


def kernel(seq, seg, token_table, position_table, segment_table):
    seq_flat = seq.reshape(-1).astype(jnp.int32)
    seg_flat = seg.reshape(-1).astype(jnp.int32)
    psg_table = (segment_table[:, None, :] + position_table[None, :, :]
                 ).reshape(_SEGS * _SENT, _HID)
    pos_flat = jnp.tile(jnp.arange(_SENT, dtype=jnp.int32), _BATCH)
    psg_idx = seg_flat * _SENT + pos_flat
    out = _emb(seq_flat, psg_idx, token_table, psg_table)
    return out.reshape(_BATCH, _SENT, _HID)


# speedup vs baseline: 1.8353x; 1.4433x over previous
import jax
import jax.numpy as jnp
from jax import lax
from jax.experimental import pallas as pl
from jax.experimental.pallas import tpu as pltpu
from jax.experimental.pallas import tpu_sc as plsc

_BATCH = 64
_SENT = 512
_HID = 768
_SEGS = 2

_N = _BATCH * _SENT
_NW = 32
_PER_W = _N // _NW
_W = 32
_CHUNKS = _PER_W // _W
_LANES = 16
_HSL = _HID // _LANES


def _emb_kernel(seq_hbm, psg_idx_hbm, tok_tab, psg_tab, out_hbm,
                idx_v, pidx_v, tok0, tok1, acc0, acc1,
                st0, st1, sp0, sp1, so0, so1):
    wid = lax.axis_index("s") * 2 + lax.axis_index("c")
    base = wid * _PER_W

    toks = (tok0, tok1)
    accs = (acc0, acc1)
    sts = (st0, st1)
    sps = (sp0, sp1)
    sos = (so0, so1)

    pltpu.sync_copy(seq_hbm.at[pl.ds(base, _PER_W)], idx_v)
    pltpu.sync_copy(psg_idx_hbm.at[pl.ds(base, _PER_W)], pidx_v)

    def gathers(b, coff):
        t = pltpu.make_async_copy(
            tok_tab.at[idx_v.at[pl.ds(coff, _W)]], toks[b], sts[b])
        p = pltpu.make_async_copy(
            psg_tab.at[pidx_v.at[pl.ds(coff, _W)]], accs[b], sps[b])
        return t, p

    def out_copy(b, coff):
        return pltpu.make_async_copy(
            accs[b], out_hbm.at[pl.ds(base + coff, _W)], sos[b])

    for b in range(2):
        t, p = gathers(b, b * _W)
        t.start()
        p.start()

    def step(it, _):
        for b in range(2):
            chunk = 2 * it + b
            coff = chunk * _W
            t, p = gathers(b, coff)
            t.wait()
            p.wait()

            def add_row(j, _):
                for h in range(_HSL):
                    sl = pl.ds(h * _LANES, _LANES)
                    plsc.addupdate(accs[b].at[j, sl], toks[b][j, sl])
                return 0

            lax.fori_loop(0, _W, add_row, 0, unroll=2)

            oc = out_copy(b, coff)
            oc.start()

            @pl.when(chunk + 2 < _CHUNKS)
            def _():
                out_copy(b, coff).wait()
                t2, p2 = gathers(b, coff + 2 * _W)
                t2.start()
                p2.start()

        return 0

    lax.fori_loop(0, _CHUNKS // 2, step, 0)

    for b in range(2):
        out_copy(b, (_CHUNKS - 2 + b) * _W).wait()


@jax.jit
def _emb(seq_flat, psg_idx, token_table, psg_table):
    mesh = plsc.VectorSubcoreMesh(core_axis_name="c", subcore_axis_name="s")
    kfn = pl.kernel(
        _emb_kernel,
        out_type=jax.ShapeDtypeStruct((_N, _HID), jnp.float32),
        mesh=mesh,
        scratch_types=[
            pltpu.VMEM((_PER_W,), jnp.int32),
            pltpu.VMEM((_PER_W,), jnp.int32),
            pltpu.VMEM((_W, _HID), jnp.float32),
            pltpu.VMEM((_W, _HID), jnp.float32),
            pltpu.VMEM((_W, _HID), jnp.float32),
            pltpu.VMEM((_W, _HID), jnp.float32),
            pltpu.SemaphoreType.DMA,
            pltpu.SemaphoreType.DMA,
            pltpu.SemaphoreType.DMA,
            pltpu.SemaphoreType.DMA,
            pltpu.SemaphoreType.DMA,
            pltpu.SemaphoreType.DMA,
        ],
    )
    return kfn(seq_flat, psg_idx, token_table, psg_table)


def kernel(seq, seg, token_table, position_table, segment_table):
    seq_flat = seq.reshape(-1).astype(jnp.int32)
    seg_flat = seg.reshape(-1).astype(jnp.int32)
    psg_table = (segment_table[:, None, :] + position_table[None, :, :]
                 ).reshape(_SEGS * _SENT, _HID)
    pos_flat = jnp.tile(jnp.arange(_SENT, dtype=jnp.int32), _BATCH)
    psg_idx = seg_flat * _SENT + pos_flat
    out = _emb(seq_flat, psg_idx, token_table, psg_table)
    return out.reshape(_BATCH, _SENT, _HID)
